# baseline (speedup 1.0000x reference)
import jax
import jax.numpy as jnp
import numpy as np
from jax.experimental import pallas as pl
from jax.experimental.pallas import tpu as pltpu
from jax.sharding import Mesh, PartitionSpec as P
from jax.experimental.shard_map import shard_map

_BM = 3072


def _vq_logits_kernel(keys_ref, emb_ref, out_ref):
    keys = keys_ref[...]
    emb = emb_ref[...]
    cross = jax.lax.dot_general(
        keys.astype(jnp.bfloat16), emb.astype(jnp.bfloat16),
        (((1,), (1,)), ((), ())),
        preferred_element_type=jnp.float32)
    k_sq = jnp.sum(keys * keys, axis=1, keepdims=True)
    e_sq = jnp.sum(emb * emb, axis=1)[None, :]
    out_ref[...] = (2.0 * cross - k_sq) - e_sq


def _vq_logits(keys, embeddings):
    B, C = keys.shape
    K = embeddings.shape[0]
    bm = min(_BM, B)
    return pl.pallas_call(
        _vq_logits_kernel,
        grid=(B // bm,),
        in_specs=[
            pl.BlockSpec((bm, C), lambda i: (i, 0)),
            pl.BlockSpec((K, C), lambda i: (0, 0)),
        ],
        out_specs=pl.BlockSpec((bm, K), lambda i: (i, 0)),
        out_shape=jax.ShapeDtypeStruct((B, K), jnp.float32),
        compiler_params=pltpu.CompilerParams(
            dimension_semantics=("parallel",)),
    )(keys, embeddings)


def kernel(keys, embeddings):
    devs = jax.devices()
    n_shards = 2 if (len(devs) >= 2 and keys.shape[0] % 2 == 0) else 1
    if n_shards == 1:
        return _vq_logits(keys, embeddings)
    mesh = Mesh(np.array(devs[:2]), ("x",))
    f = shard_map(
        _vq_logits, mesh=mesh,
        in_specs=(P("x", None), P(None, None)),
        out_specs=P("x", None),
        check_rep=False,
    )
    return f(keys, embeddings)


# speedup vs baseline: 10.4397x; 10.4397x over previous
import jax
import jax.numpy as jnp
from jax.experimental import pallas as pl
from jax.experimental.pallas import tpu as pltpu

_SLAB = 512
_NBUF = 8


def _vq_logits_kernel(keys_ref, emb_ref, out_ref, stage_ref, sems):
    emb = emb_ref[...]
    emb_bf = emb.astype(jnp.bfloat16)
    e_sq = jnp.sum(emb * emb, axis=1)[None, :]
    nslab = keys_ref.shape[0] // _SLAB

    def slab_copy(i, j):
        return pltpu.make_async_copy(
            stage_ref.at[j],
            out_ref.at[pl.ds(i * _SLAB, _SLAB), :],
            sems.at[j])

    def body(i, carry):
        j = jax.lax.rem(i, _NBUF)

        @pl.when(i >= _NBUF)
        def _():
            slab_copy(i - _NBUF, j).wait()

        keys = keys_ref[pl.ds(i * _SLAB, _SLAB), :]
        cross = jax.lax.dot_general(
            keys.astype(jnp.bfloat16), emb_bf,
            (((1,), (1,)), ((), ())),
            preferred_element_type=jnp.float32)
        k_sq = jnp.sum(keys * keys, axis=1, keepdims=True)
        stage_ref[j] = (2.0 * cross - k_sq) - e_sq
        slab_copy(i, j).start()
        return carry

    jax.lax.fori_loop(0, nslab, body, 0)

    def drain(i, carry):
        slab_copy(i, jax.lax.rem(i, _NBUF)).wait()
        return carry

    jax.lax.fori_loop(jnp.maximum(0, nslab - _NBUF), nslab, drain, 0)


def kernel(keys, embeddings):
    B, C = keys.shape
    K = embeddings.shape[0]
    return pl.pallas_call(
        _vq_logits_kernel,
        in_specs=[
            pl.BlockSpec(memory_space=pltpu.MemorySpace.VMEM),
            pl.BlockSpec(memory_space=pltpu.MemorySpace.VMEM),
        ],
        out_specs=pl.BlockSpec(memory_space=pl.ANY),
        out_shape=jax.ShapeDtypeStruct((B, K), jnp.float32),
        scratch_shapes=[
            pltpu.VMEM((_NBUF, _SLAB, K), jnp.float32),
            pltpu.SemaphoreType.DMA((_NBUF,)),
        ],
    )(keys, embeddings)
